# Initial kernel scaffold; baseline (speedup 1.0000x reference)
import jax
import jax.numpy as jnp
from jax.experimental import pallas as pl


def kernel(feat_src, feat_dst, edge_index, W_src, b_src, W_dst, b_dst, attn_src):
    raise NotImplementedError("write your pallas kernel here")


# speedup vs baseline: 68.3461x; 68.3461x over previous
import functools

import jax
import jax.numpy as jnp
import numpy as np
from jax import lax
from jax.experimental import pallas as pl
from jax.experimental.pallas import tpu as pltpu
from jax.experimental.pallas import tpu_sc as plsc

H = 8
O = 16
D = 128
N = 10000
E = 320000
NEG_SLOPE = 0.2

EXT = 144
NW = 32
EPW = E // NW
CH = 80
NCHUNK = EPW // CH
ROWS_PER_TILE = N // 16
ZROWS = 125


def _tc_pre_body(x1_ref, x2_ref, w1_ref, b1_ref, w2_ref, b2_ref,
                 al_ref, ar_ref, g_ref, out1_ref, out2_ref):
    h1 = jnp.dot(x1_ref[...], w1_ref[...],
                 preferred_element_type=jnp.float32) + b1_ref[...]
    el = jnp.dot(h1 * al_ref[...], g_ref[...],
                 preferred_element_type=jnp.float32)
    out1_ref[:, :D] = h1
    out1_ref[:, D:] = el
    h2 = jnp.dot(x2_ref[...], w2_ref[...],
                 preferred_element_type=jnp.float32) + b2_ref[...]
    out2_ref[...] = jnp.dot(h2 * ar_ref[...], g_ref[...],
                            preferred_element_type=jnp.float32)


def _tc_post_body(acc_ref, r_ref, out_ref):
    a = acc_ref[0] + acc_ref[1]
    den = a[:, D:D + H]
    inv = jnp.where(den > 0.0, 1.0 / den, 0.0)
    out_ref[...] = a[:, :D] * jnp.dot(inv, r_ref[...],
                                      preferred_element_type=jnp.float32)


def _sc_edge_body(hs_hbm, er_hbm, eidx_hbm, out_hbm,
                  src_v, dst_v, hs_g, er_g, zbuf, acc_sh, sem1, sem2):
    cid = lax.axis_index("c")
    sid = lax.axis_index("s")
    wid = sid * 2 + cid

    def zb(i, _):
        r = i // (EXT // 16)
        k = i - r * (EXT // 16)
        zbuf[r, pl.ds(k * 16, 16)] = jnp.zeros((16,), jnp.float32)
        return 0
    lax.fori_loop(0, ZROWS * (EXT // 16), zb, 0)

    def zcp(k, _):
        pltpu.sync_copy(
            zbuf, acc_sh.at[pl.ds(sid * ROWS_PER_TILE + k * ZROWS, ZROWS)])
        return 0
    lax.fori_loop(0, ROWS_PER_TILE // ZROWS, zcp, 0)
    plsc.subcore_barrier()

    def chunk(i, _):
        base = wid * EPW + i * CH
        pltpu.sync_copy(eidx_hbm.at[0, pl.ds(base, CH)], src_v)
        pltpu.sync_copy(eidx_hbm.at[1, pl.ds(base, CH)], dst_v)
        cp1 = pltpu.async_copy(hs_hbm.at[src_v], hs_g, sem1)
        cp2 = pltpu.async_copy(er_hbm.at[dst_v], er_g, sem2)
        cp1.wait()
        cp2.wait()

        def edge(c, _):
            ev = hs_g[c, pl.ds(D, 16)] + er_g[c, :]
            ev = jnp.where(ev >= 0.0, ev, NEG_SLOPE * ev)
            sv = jnp.exp(ev)
            hs_g[c, pl.ds(D, 16)] = sv
            for j in range(H):
                hs_g[c, pl.ds(j * 16, 16)] = (
                    hs_g[c, pl.ds(j * 16, 16)] * jnp.full((16,), sv[j], jnp.float32))
            return 0
        lax.fori_loop(0, CH, edge, 0)
        pltpu.sync_copy(hs_g, acc_sh.at[dst_v], add=True)
        return 0
    lax.fori_loop(0, NCHUNK, chunk, 0)

    plsc.subcore_barrier()
    pltpu.sync_copy(acc_sh.at[pl.ds(sid * ROWS_PER_TILE, ROWS_PER_TILE)],
                    out_hbm.at[cid, pl.ds(sid * ROWS_PER_TILE, ROWS_PER_TILE)])


def kernel(feat_src, feat_dst, edge_index, W_src, b_src, W_dst, b_dst, attn_src):
    f32 = jnp.float32
    g_np = np.zeros((D, 16), np.float32)
    r_np = np.zeros((H, D), np.float32)
    for h in range(H):
        g_np[h * O:(h + 1) * O, h] = 1.0
        r_np[h, h * O:(h + 1) * O] = 1.0
    G = jnp.asarray(g_np)
    R = jnp.asarray(r_np)
    attn_l = attn_src[:, :O].reshape(1, D)
    attn_r = attn_src[:, O:].reshape(1, D)
    b1 = b_src.reshape(1, D)
    b2 = b_dst.reshape(1, D)

    blk = 1000
    grid = (N // blk,)
    hs_ext, er_pad = pl.pallas_call(
        _tc_pre_body,
        grid=grid,
        in_specs=[
            pl.BlockSpec((blk, D), lambda i: (i, 0)),
            pl.BlockSpec((blk, D), lambda i: (i, 0)),
            pl.BlockSpec((D, D), lambda i: (0, 0)),
            pl.BlockSpec((1, D), lambda i: (0, 0)),
            pl.BlockSpec((D, D), lambda i: (0, 0)),
            pl.BlockSpec((1, D), lambda i: (0, 0)),
            pl.BlockSpec((1, D), lambda i: (0, 0)),
            pl.BlockSpec((1, D), lambda i: (0, 0)),
            pl.BlockSpec((D, 16), lambda i: (0, 0)),
        ],
        out_specs=[
            pl.BlockSpec((blk, EXT), lambda i: (i, 0)),
            pl.BlockSpec((blk, 16), lambda i: (i, 0)),
        ],
        out_shape=[
            jax.ShapeDtypeStruct((N, EXT), f32),
            jax.ShapeDtypeStruct((N, 16), f32),
        ],
    )(feat_src, feat_dst, W_src, b1, W_dst, b2, attn_l, attn_r, G)

    mesh = plsc.VectorSubcoreMesh(core_axis_name="c", subcore_axis_name="s")
    sc_edge = functools.partial(
        pl.kernel,
        mesh=mesh,
        compiler_params=pltpu.CompilerParams(use_tc_tiling_on_sc=False),
        out_type=jax.ShapeDtypeStruct((2, N, EXT), f32),
        scratch_types=[
            pltpu.VMEM((CH,), jnp.int32),
            pltpu.VMEM((CH,), jnp.int32),
            pltpu.VMEM((CH, EXT), f32),
            pltpu.VMEM((CH, 16), f32),
            pltpu.VMEM((ZROWS, EXT), f32),
            pltpu.VMEM_SHARED((N, EXT), f32),
            pltpu.SemaphoreType.DMA,
            pltpu.SemaphoreType.DMA,
        ],
    )(_sc_edge_body)
    acc = sc_edge(hs_ext, er_pad, edge_index)

    out = pl.pallas_call(
        _tc_post_body,
        grid=grid,
        in_specs=[
            pl.BlockSpec((2, blk, EXT), lambda i: (0, i, 0)),
            pl.BlockSpec((H, D), lambda i: (0, 0)),
        ],
        out_specs=pl.BlockSpec((blk, D), lambda i: (i, 0)),
        out_shape=jax.ShapeDtypeStruct((N, D), f32),
    )(acc, R)
    return out


# speedup vs baseline: 97.1772x; 1.4218x over previous
import functools

import jax
import jax.numpy as jnp
import numpy as np
from jax import lax
from jax.experimental import pallas as pl
from jax.experimental.pallas import tpu as pltpu
from jax.experimental.pallas import tpu_sc as plsc

H = 8
O = 16
D = 128
N = 10000
E = 320000
NEG_SLOPE = 0.2

EXT = 144
NW = 32
EPW = E // NW
CH = 80
NCHUNK = EPW // CH
ROWS_PER_TILE = N // 16
ZROWS = 25


def _tc_pre_body(x1_ref, x2_ref, w1_ref, b1_ref, w2_ref, b2_ref,
                 al_ref, ar_ref, g_ref, out1_ref, out2_ref):
    h1 = jnp.dot(x1_ref[...], w1_ref[...],
                 preferred_element_type=jnp.float32) + b1_ref[...]
    el = jnp.dot(h1 * al_ref[...], g_ref[...],
                 preferred_element_type=jnp.float32)
    out1_ref[:, :D] = h1
    out1_ref[:, D:] = el
    h2 = jnp.dot(x2_ref[...], w2_ref[...],
                 preferred_element_type=jnp.float32) + b2_ref[...]
    out2_ref[...] = jnp.dot(h2 * ar_ref[...], g_ref[...],
                            preferred_element_type=jnp.float32)


def _tc_post_body(acc_ref, r_ref, out_ref):
    a = acc_ref[0] + acc_ref[1]
    den = a[:, D:D + H]
    inv = jnp.where(den > 0.0, 1.0 / den, 0.0)
    out_ref[...] = a[:, :D] * jnp.dot(inv, r_ref[...],
                                      preferred_element_type=jnp.float32)


def _sc_edge_body(hs_hbm, er_hbm, eidx_hbm, out_hbm,
                  src_a, dst_a, hs_a, er_a, src_b, dst_b, hs_b, er_b,
                  zbuf, acc_sh, isem_a, isem_b, gsem_a, gsem_b):
    cid = lax.axis_index("c")
    sid = lax.axis_index("s")
    wid = sid * 2 + cid
    ebase = wid * EPW

    def zb(i, _):
        r = i // (EXT // 16)
        k = i - r * (EXT // 16)
        zbuf[r, pl.ds(k * 16, 16)] = jnp.zeros((16,), jnp.float32)
        return 0
    lax.fori_loop(0, ZROWS * (EXT // 16), zb, 0)

    def zcp(k, _):
        pltpu.sync_copy(
            zbuf, acc_sh.at[pl.ds(sid * ROWS_PER_TILE + k * ZROWS, ZROWS)])
        return 0
    lax.fori_loop(0, ROWS_PER_TILE // ZROWS, zcp, 0)
    plsc.subcore_barrier()

    def start_idx(e, sv, dv, isem):
        b = ebase + e * CH
        pltpu.async_copy(eidx_hbm.at[0, pl.ds(b, CH)], sv, isem)
        pltpu.async_copy(eidx_hbm.at[1, pl.ds(b, CH)], dv, isem)

    def drain_idx(sv, dv, isem):
        pltpu.make_async_copy(eidx_hbm.at[0, pl.ds(0, CH)], sv, isem).wait()
        pltpu.make_async_copy(eidx_hbm.at[1, pl.ds(0, CH)], dv, isem).wait()

    def start_gather(sv, dv, hsb, erb, gsem):
        pltpu.async_copy(hs_hbm.at[sv], hsb, gsem)
        pltpu.async_copy(er_hbm.at[dv], erb, gsem)

    def drain_gather(hsb, erb, gsem):
        pltpu.make_async_copy(hs_hbm.at[pl.ds(0, CH)], hsb, gsem).wait()
        pltpu.make_async_copy(er_hbm.at[pl.ds(0, CH)], erb, gsem).wait()

    def compute(hsb, erb):
        def edge2(k, _):
            for t in range(2):
                c = 2 * k + t
                ev = hsb[c, pl.ds(D, 16)] + erb[c, :]
                ev = jnp.where(ev >= 0.0, ev, NEG_SLOPE * ev)
                sv = jnp.exp(ev)
                hsb[c, pl.ds(D, 16)] = sv
                for j in range(H):
                    hsb[c, pl.ds(j * 16, 16)] = (
                        hsb[c, pl.ds(j * 16, 16)]
                        * jnp.full((16,), sv[j], jnp.float32))
            return 0
        lax.fori_loop(0, CH // 2, edge2, 0)

    def process(e, cur, nxt):
        (src_c, dst_c, hs_c, er_c, isem_c, gsem_c) = cur
        (src_n, dst_n, hs_n, er_n, isem_n, gsem_n) = nxt
        drain_idx(src_n, dst_n, isem_n)
        start_gather(src_n, dst_n, hs_n, er_n, gsem_n)
        drain_gather(hs_c, er_c, gsem_c)
        compute(hs_c, er_c)
        pltpu.sync_copy(hs_c, acc_sh.at[dst_c], add=True)
        @pl.when(e + 2 < NCHUNK)
        def _():
            start_idx(e + 2, src_c, dst_c, isem_c)

    buf_a = (src_a, dst_a, hs_a, er_a, isem_a, gsem_a)
    buf_b = (src_b, dst_b, hs_b, er_b, isem_b, gsem_b)

    start_idx(0, src_a, dst_a, isem_a)
    start_idx(1, src_b, dst_b, isem_b)
    drain_idx(src_a, dst_a, isem_a)
    start_gather(src_a, dst_a, hs_a, er_a, gsem_a)

    def pair(j, _):
        process(2 * j, buf_a, buf_b)
        process(2 * j + 1, buf_b, buf_a)
        return 0
    lax.fori_loop(0, NCHUNK // 2, pair, 0)

    drain_gather(hs_a, er_a, gsem_a)
    compute(hs_a, er_a)
    pltpu.sync_copy(hs_a, acc_sh.at[dst_a], add=True)

    plsc.subcore_barrier()
    pltpu.sync_copy(acc_sh.at[pl.ds(sid * ROWS_PER_TILE, ROWS_PER_TILE)],
                    out_hbm.at[cid, pl.ds(sid * ROWS_PER_TILE, ROWS_PER_TILE)])


def kernel(feat_src, feat_dst, edge_index, W_src, b_src, W_dst, b_dst, attn_src):
    f32 = jnp.float32
    g_np = np.zeros((D, 16), np.float32)
    r_np = np.zeros((H, D), np.float32)
    for h in range(H):
        g_np[h * O:(h + 1) * O, h] = 1.0
        r_np[h, h * O:(h + 1) * O] = 1.0
    G = jnp.asarray(g_np)
    R = jnp.asarray(r_np)
    attn_l = attn_src[:, :O].reshape(1, D)
    attn_r = attn_src[:, O:].reshape(1, D)
    b1 = b_src.reshape(1, D)
    b2 = b_dst.reshape(1, D)

    blk = 1000
    grid = (N // blk,)
    hs_ext, er_pad = pl.pallas_call(
        _tc_pre_body,
        grid=grid,
        in_specs=[
            pl.BlockSpec((blk, D), lambda i: (i, 0)),
            pl.BlockSpec((blk, D), lambda i: (i, 0)),
            pl.BlockSpec((D, D), lambda i: (0, 0)),
            pl.BlockSpec((1, D), lambda i: (0, 0)),
            pl.BlockSpec((D, D), lambda i: (0, 0)),
            pl.BlockSpec((1, D), lambda i: (0, 0)),
            pl.BlockSpec((1, D), lambda i: (0, 0)),
            pl.BlockSpec((1, D), lambda i: (0, 0)),
            pl.BlockSpec((D, 16), lambda i: (0, 0)),
        ],
        out_specs=[
            pl.BlockSpec((blk, EXT), lambda i: (i, 0)),
            pl.BlockSpec((blk, 16), lambda i: (i, 0)),
        ],
        out_shape=[
            jax.ShapeDtypeStruct((N, EXT), f32),
            jax.ShapeDtypeStruct((N, 16), f32),
        ],
    )(feat_src, feat_dst, W_src, b1, W_dst, b2, attn_l, attn_r, G)

    mesh = plsc.VectorSubcoreMesh(core_axis_name="c", subcore_axis_name="s")
    sc_edge = functools.partial(
        pl.kernel,
        mesh=mesh,
        compiler_params=pltpu.CompilerParams(use_tc_tiling_on_sc=False),
        out_type=jax.ShapeDtypeStruct((2, N, EXT), f32),
        scratch_types=[
            pltpu.VMEM((CH,), jnp.int32),
            pltpu.VMEM((CH,), jnp.int32),
            pltpu.VMEM((CH, EXT), f32),
            pltpu.VMEM((CH, 16), f32),
            pltpu.VMEM((CH,), jnp.int32),
            pltpu.VMEM((CH,), jnp.int32),
            pltpu.VMEM((CH, EXT), f32),
            pltpu.VMEM((CH, 16), f32),
            pltpu.VMEM((ZROWS, EXT), f32),
            pltpu.VMEM_SHARED((N, EXT), f32),
            pltpu.SemaphoreType.DMA,
            pltpu.SemaphoreType.DMA,
            pltpu.SemaphoreType.DMA,
            pltpu.SemaphoreType.DMA,
        ],
    )(_sc_edge_body)
    acc = sc_edge(hs_ext, er_pad, edge_index)

    out = pl.pallas_call(
        _tc_post_body,
        grid=grid,
        in_specs=[
            pl.BlockSpec((2, blk, EXT), lambda i: (0, i, 0)),
            pl.BlockSpec((H, D), lambda i: (0, 0)),
        ],
        out_specs=pl.BlockSpec((blk, D), lambda i: (i, 0)),
        out_shape=jax.ShapeDtypeStruct((N, D), f32),
    )(acc, R)
    return out


# speedup vs baseline: 102.0487x; 1.0501x over previous
import functools

import jax
import jax.numpy as jnp
import numpy as np
from jax import lax
from jax.experimental import pallas as pl
from jax.experimental.pallas import tpu as pltpu
from jax.experimental.pallas import tpu_sc as plsc

H = 8
O = 16
D = 128
N = 10000
E = 320000
NEG_SLOPE = 0.2

EXT = 144
NW = 32
EPW = E // NW
CH = 80
NCHUNK = EPW // CH
ROWS_PER_TILE = N // 16
ZROWS = 25


def _tc_pre_body(x1_ref, x2_ref, w1_ref, b1_ref, w2_ref, b2_ref,
                 al_ref, ar_ref, g_ref, out1_ref, out2_ref):
    h1 = jnp.dot(x1_ref[...], w1_ref[...],
                 preferred_element_type=jnp.float32) + b1_ref[...]
    el = jnp.dot(h1 * al_ref[...], g_ref[...],
                 preferred_element_type=jnp.float32)
    out1_ref[:, :D] = h1
    out1_ref[:, D:] = el
    h2 = jnp.dot(x2_ref[...], w2_ref[...],
                 preferred_element_type=jnp.float32) + b2_ref[...]
    out2_ref[...] = jnp.dot(h2 * ar_ref[...], g_ref[...],
                            preferred_element_type=jnp.float32)


def _tc_post_body(acc_ref, r_ref, out_ref):
    a = acc_ref[0] + acc_ref[1]
    den = a[:, D:D + H]
    inv = jnp.where(den > 0.0, 1.0 / den, 0.0)
    out_ref[...] = a[:, :D] * jnp.dot(inv, r_ref[...],
                                      preferred_element_type=jnp.float32)


def _sc_edge_body(hs_hbm, er_hbm, eidx_hbm, out_hbm,
                  src_a, dst_a, hs_a, er_a, src_b, dst_b, hs_b, er_b,
                  zbuf, acc_sh, isem_a, isem_b, gsem_a, gsem_b):
    cid = lax.axis_index("c")
    sid = lax.axis_index("s")
    wid = sid * 2 + cid
    ebase = wid * EPW

    def zb(i, _):
        r = i // (EXT // 16)
        k = i - r * (EXT // 16)
        zbuf[r, pl.ds(k * 16, 16)] = jnp.zeros((16,), jnp.float32)
        return 0
    lax.fori_loop(0, ZROWS * (EXT // 16), zb, 0)

    def zcp(k, _):
        pltpu.sync_copy(
            zbuf, acc_sh.at[pl.ds(sid * ROWS_PER_TILE + k * ZROWS, ZROWS)])
        return 0
    lax.fori_loop(0, ROWS_PER_TILE // ZROWS, zcp, 0)
    plsc.subcore_barrier()

    def start_idx(e, sv, dv, isem):
        b = ebase + e * CH
        pltpu.async_copy(eidx_hbm.at[0, pl.ds(b, CH)], sv, isem)
        pltpu.async_copy(eidx_hbm.at[1, pl.ds(b, CH)], dv, isem)

    def drain_idx(sv, dv, isem):
        pltpu.make_async_copy(eidx_hbm.at[0, pl.ds(0, CH)], sv, isem).wait()
        pltpu.make_async_copy(eidx_hbm.at[1, pl.ds(0, CH)], dv, isem).wait()

    def start_gather(sv, dv, hsb, erb, gsem):
        pltpu.async_copy(hs_hbm.at[sv], hsb, gsem)
        pltpu.async_copy(er_hbm.at[dv], erb, gsem)

    def drain_gather(hsb, erb, gsem):
        pltpu.make_async_copy(hs_hbm.at[pl.ds(0, CH)], hsb, gsem).wait()
        pltpu.make_async_copy(er_hbm.at[pl.ds(0, CH)], erb, gsem).wait()

    def compute(hsb, erb):
        def spass(k, _):
            for t in range(4):
                c = 4 * k + t
                ev = hsb[c, pl.ds(D, 16)] + erb[c, :]
                ev = jnp.where(ev >= 0.0, ev, NEG_SLOPE * ev)
                hsb[c, pl.ds(D, 16)] = jnp.exp(ev)
            return 0
        lax.fori_loop(0, CH // 4, spass, 0)

        def mpass(k, _):
            for t in range(2):
                c = 2 * k + t
                sv = hsb[c, pl.ds(D, 16)]
                for j in range(H):
                    hsb[c, pl.ds(j * 16, 16)] = (
                        hsb[c, pl.ds(j * 16, 16)]
                        * jnp.full((16,), sv[j], jnp.float32))
            return 0
        lax.fori_loop(0, CH // 2, mpass, 0)

    def process(e, cur, nxt):
        (src_c, dst_c, hs_c, er_c, isem_c, gsem_c) = cur
        (src_n, dst_n, hs_n, er_n, isem_n, gsem_n) = nxt
        drain_idx(src_n, dst_n, isem_n)
        start_gather(src_n, dst_n, hs_n, er_n, gsem_n)
        drain_gather(hs_c, er_c, gsem_c)
        compute(hs_c, er_c)
        pltpu.sync_copy(hs_c, acc_sh.at[dst_c], add=True)
        @pl.when(e + 2 < NCHUNK)
        def _():
            start_idx(e + 2, src_c, dst_c, isem_c)

    buf_a = (src_a, dst_a, hs_a, er_a, isem_a, gsem_a)
    buf_b = (src_b, dst_b, hs_b, er_b, isem_b, gsem_b)

    start_idx(0, src_a, dst_a, isem_a)
    start_idx(1, src_b, dst_b, isem_b)
    drain_idx(src_a, dst_a, isem_a)
    start_gather(src_a, dst_a, hs_a, er_a, gsem_a)

    def pair(j, _):
        process(2 * j, buf_a, buf_b)
        process(2 * j + 1, buf_b, buf_a)
        return 0
    lax.fori_loop(0, NCHUNK // 2, pair, 0)

    drain_gather(hs_a, er_a, gsem_a)
    compute(hs_a, er_a)
    pltpu.sync_copy(hs_a, acc_sh.at[dst_a], add=True)

    plsc.subcore_barrier()
    pltpu.sync_copy(acc_sh.at[pl.ds(sid * ROWS_PER_TILE, ROWS_PER_TILE)],
                    out_hbm.at[cid, pl.ds(sid * ROWS_PER_TILE, ROWS_PER_TILE)])


def kernel(feat_src, feat_dst, edge_index, W_src, b_src, W_dst, b_dst, attn_src):
    f32 = jnp.float32
    g_np = np.zeros((D, 16), np.float32)
    r_np = np.zeros((H, D), np.float32)
    for h in range(H):
        g_np[h * O:(h + 1) * O, h] = 1.0
        r_np[h, h * O:(h + 1) * O] = 1.0
    G = jnp.asarray(g_np)
    R = jnp.asarray(r_np)
    attn_l = attn_src[:, :O].reshape(1, D)
    attn_r = attn_src[:, O:].reshape(1, D)
    b1 = b_src.reshape(1, D)
    b2 = b_dst.reshape(1, D)

    blk = 1000
    grid = (N // blk,)
    hs_ext, er_pad = pl.pallas_call(
        _tc_pre_body,
        grid=grid,
        in_specs=[
            pl.BlockSpec((blk, D), lambda i: (i, 0)),
            pl.BlockSpec((blk, D), lambda i: (i, 0)),
            pl.BlockSpec((D, D), lambda i: (0, 0)),
            pl.BlockSpec((1, D), lambda i: (0, 0)),
            pl.BlockSpec((D, D), lambda i: (0, 0)),
            pl.BlockSpec((1, D), lambda i: (0, 0)),
            pl.BlockSpec((1, D), lambda i: (0, 0)),
            pl.BlockSpec((1, D), lambda i: (0, 0)),
            pl.BlockSpec((D, 16), lambda i: (0, 0)),
        ],
        out_specs=[
            pl.BlockSpec((blk, EXT), lambda i: (i, 0)),
            pl.BlockSpec((blk, 16), lambda i: (i, 0)),
        ],
        out_shape=[
            jax.ShapeDtypeStruct((N, EXT), f32),
            jax.ShapeDtypeStruct((N, 16), f32),
        ],
    )(feat_src, feat_dst, W_src, b1, W_dst, b2, attn_l, attn_r, G)

    mesh = plsc.VectorSubcoreMesh(core_axis_name="c", subcore_axis_name="s")
    sc_edge = functools.partial(
        pl.kernel,
        mesh=mesh,
        compiler_params=pltpu.CompilerParams(use_tc_tiling_on_sc=False),
        out_type=jax.ShapeDtypeStruct((2, N, EXT), f32),
        scratch_types=[
            pltpu.VMEM((CH,), jnp.int32),
            pltpu.VMEM((CH,), jnp.int32),
            pltpu.VMEM((CH, EXT), f32),
            pltpu.VMEM((CH, 16), f32),
            pltpu.VMEM((CH,), jnp.int32),
            pltpu.VMEM((CH,), jnp.int32),
            pltpu.VMEM((CH, EXT), f32),
            pltpu.VMEM((CH, 16), f32),
            pltpu.VMEM((ZROWS, EXT), f32),
            pltpu.VMEM_SHARED((N, EXT), f32),
            pltpu.SemaphoreType.DMA,
            pltpu.SemaphoreType.DMA,
            pltpu.SemaphoreType.DMA,
            pltpu.SemaphoreType.DMA,
        ],
    )(_sc_edge_body)
    acc = sc_edge(hs_ext, er_pad, edge_index)

    out = pl.pallas_call(
        _tc_post_body,
        grid=grid,
        in_specs=[
            pl.BlockSpec((2, blk, EXT), lambda i: (0, i, 0)),
            pl.BlockSpec((H, D), lambda i: (0, 0)),
        ],
        out_specs=pl.BlockSpec((blk, D), lambda i: (i, 0)),
        out_shape=jax.ShapeDtypeStruct((N, D), f32),
    )(acc, R)
    return out


# speedup vs baseline: 118.5206x; 1.1614x over previous
import functools

import jax
import jax.numpy as jnp
import numpy as np
from jax import lax
from jax.experimental import pallas as pl
from jax.experimental.pallas import tpu as pltpu
from jax.experimental.pallas import tpu_sc as plsc

H = 8
O = 16
D = 128
N = 10000
E = 320000
NEG_SLOPE = 0.2

EXT = 144
NW = 32
EPW = E // NW
CH = 80
NCHUNK = EPW // CH
ROWS_PER_TILE = N // 16
ZROWS = 25


def _tc_pre_body(x1_ref, x2_ref, w1_ref, b1_ref, w2_ref, b2_ref,
                 al_ref, ar_ref, g_ref, out1_ref, out2_ref):
    h1 = jnp.dot(x1_ref[...], w1_ref[...],
                 preferred_element_type=jnp.float32) + b1_ref[...]
    el = jnp.dot(h1 * al_ref[...], g_ref[...],
                 preferred_element_type=jnp.float32)
    out1_ref[:, :D] = h1
    out1_ref[:, D:] = el
    h2 = jnp.dot(x2_ref[...], w2_ref[...],
                 preferred_element_type=jnp.float32) + b2_ref[...]
    out2_ref[...] = jnp.dot(h2 * ar_ref[...], g_ref[...],
                            preferred_element_type=jnp.float32)


def _tc_post_body(acc_ref, r_ref, out_ref):
    a = acc_ref[0] + acc_ref[1]
    den = a[:, D:D + H]
    inv = jnp.where(den > 0.0, 1.0 / den, 0.0)
    out_ref[...] = a[:, :D] * jnp.dot(inv, r_ref[...],
                                      preferred_element_type=jnp.float32)


def _sc_edge_body(hs_hbm, er_hbm, eidx_hbm, out_hbm,
                  src_a, dst_a, hs_a, er_a, src_b, dst_b, hs_b, er_b,
                  scat_a, scat_b, zbuf, acc_sh,
                  isem_a, isem_b, gsem_a, gsem_b, ssem_a, ssem_b):
    cid = lax.axis_index("c")
    sid = lax.axis_index("s")
    wid = sid * 2 + cid
    ebase = wid * EPW

    def zb(i, _):
        r = i // (EXT // 16)
        k = i - r * (EXT // 16)
        zbuf[r, pl.ds(k * 16, 16)] = jnp.zeros((16,), jnp.float32)
        return 0
    lax.fori_loop(0, ZROWS * (EXT // 16), zb, 0)

    def zcp(k, _):
        pltpu.sync_copy(
            zbuf, acc_sh.at[pl.ds(sid * ROWS_PER_TILE + k * ZROWS, ZROWS)])
        return 0
    lax.fori_loop(0, ROWS_PER_TILE // ZROWS, zcp, 0)
    plsc.subcore_barrier()

    def start_idx(e, sv, dv, isem):
        b = ebase + e * CH
        pltpu.async_copy(eidx_hbm.at[0, pl.ds(b, CH)], sv, isem)
        pltpu.async_copy(eidx_hbm.at[1, pl.ds(b, CH)], dv, isem)

    def drain_idx(sv, dv, isem):
        pltpu.make_async_copy(eidx_hbm.at[0, pl.ds(0, CH)], sv, isem).wait()
        pltpu.make_async_copy(eidx_hbm.at[1, pl.ds(0, CH)], dv, isem).wait()

    def start_gather(sv, dv, hsb, erb, gsem):
        pltpu.async_copy(hs_hbm.at[sv], hsb, gsem)
        pltpu.async_copy(er_hbm.at[dv], erb, gsem)

    def drain_gather(hsb, erb, gsem):
        pltpu.make_async_copy(hs_hbm.at[pl.ds(0, CH)], hsb, gsem).wait()
        pltpu.make_async_copy(er_hbm.at[pl.ds(0, CH)], erb, gsem).wait()

    def compute(hsb, erb):
        def spass(k, _):
            for t in range(4):
                c = 4 * k + t
                ev = hsb[c, pl.ds(D, 16)] + erb[c, :]
                ev = jnp.where(ev >= 0.0, ev, NEG_SLOPE * ev)
                hsb[c, pl.ds(D, 16)] = jnp.exp(ev)
            return 0
        lax.fori_loop(0, CH // 4, spass, 0)

        def mpass(k, _):
            for t in range(2):
                c = 2 * k + t
                sv = hsb[c, pl.ds(D, 16)]
                for j in range(H):
                    hsb[c, pl.ds(j * 16, 16)] = (
                        hsb[c, pl.ds(j * 16, 16)]
                        * jnp.full((16,), sv[j], jnp.float32))
            return 0
        lax.fori_loop(0, CH // 2, mpass, 0)

    def copy_scat_idx(dv, sv_scat):
        for t in range(CH // 16):
            sv_scat[pl.ds(t * 16, 16)] = dv[pl.ds(t * 16, 16)]

    def drain_scatter(hsb, scat, ssem):
        pltpu.make_async_copy(hsb, acc_sh.at[scat], ssem).wait()

    def process(e, cur, nxt):
        (src_c, dst_c, hs_c, er_c, scat_c, isem_c, gsem_c, ssem_c) = cur
        (src_n, dst_n, hs_n, er_n, scat_n, isem_n, gsem_n, ssem_n) = nxt
        drain_idx(src_n, dst_n, isem_n)

        @pl.when(e > 0)
        def _():
            drain_scatter(hs_n, scat_n, ssem_n)
        start_gather(src_n, dst_n, hs_n, er_n, gsem_n)
        drain_gather(hs_c, er_c, gsem_c)
        copy_scat_idx(dst_c, scat_c)

        @pl.when(e + 2 < NCHUNK)
        def _():
            start_idx(e + 2, src_c, dst_c, isem_c)
        compute(hs_c, er_c)
        pltpu.async_copy(hs_c, acc_sh.at[scat_c], ssem_c, add=True)

    buf_a = (src_a, dst_a, hs_a, er_a, scat_a, isem_a, gsem_a, ssem_a)
    buf_b = (src_b, dst_b, hs_b, er_b, scat_b, isem_b, gsem_b, ssem_b)

    start_idx(0, src_a, dst_a, isem_a)
    start_idx(1, src_b, dst_b, isem_b)
    drain_idx(src_a, dst_a, isem_a)
    start_gather(src_a, dst_a, hs_a, er_a, gsem_a)

    def pair(j, _):
        process(2 * j, buf_a, buf_b)
        process(2 * j + 1, buf_b, buf_a)
        return 0
    lax.fori_loop(0, NCHUNK // 2, pair, 0)

    drain_scatter(hs_b, scat_b, ssem_b)
    drain_gather(hs_a, er_a, gsem_a)
    copy_scat_idx(dst_a, scat_a)
    compute(hs_a, er_a)
    pltpu.async_copy(hs_a, acc_sh.at[scat_a], ssem_a, add=True)
    drain_scatter(hs_a, scat_a, ssem_a)

    plsc.subcore_barrier()
    pltpu.sync_copy(acc_sh.at[pl.ds(sid * ROWS_PER_TILE, ROWS_PER_TILE)],
                    out_hbm.at[cid, pl.ds(sid * ROWS_PER_TILE, ROWS_PER_TILE)])


def kernel(feat_src, feat_dst, edge_index, W_src, b_src, W_dst, b_dst, attn_src):
    f32 = jnp.float32
    g_np = np.zeros((D, 16), np.float32)
    r_np = np.zeros((H, D), np.float32)
    for h in range(H):
        g_np[h * O:(h + 1) * O, h] = 1.0
        r_np[h, h * O:(h + 1) * O] = 1.0
    G = jnp.asarray(g_np)
    R = jnp.asarray(r_np)
    attn_l = attn_src[:, :O].reshape(1, D)
    attn_r = attn_src[:, O:].reshape(1, D)
    b1 = b_src.reshape(1, D)
    b2 = b_dst.reshape(1, D)

    blk = 1000
    grid = (N // blk,)
    hs_ext, er_pad = pl.pallas_call(
        _tc_pre_body,
        grid=grid,
        in_specs=[
            pl.BlockSpec((blk, D), lambda i: (i, 0)),
            pl.BlockSpec((blk, D), lambda i: (i, 0)),
            pl.BlockSpec((D, D), lambda i: (0, 0)),
            pl.BlockSpec((1, D), lambda i: (0, 0)),
            pl.BlockSpec((D, D), lambda i: (0, 0)),
            pl.BlockSpec((1, D), lambda i: (0, 0)),
            pl.BlockSpec((1, D), lambda i: (0, 0)),
            pl.BlockSpec((1, D), lambda i: (0, 0)),
            pl.BlockSpec((D, 16), lambda i: (0, 0)),
        ],
        out_specs=[
            pl.BlockSpec((blk, EXT), lambda i: (i, 0)),
            pl.BlockSpec((blk, 16), lambda i: (i, 0)),
        ],
        out_shape=[
            jax.ShapeDtypeStruct((N, EXT), f32),
            jax.ShapeDtypeStruct((N, 16), f32),
        ],
    )(feat_src, feat_dst, W_src, b1, W_dst, b2, attn_l, attn_r, G)

    mesh = plsc.VectorSubcoreMesh(core_axis_name="c", subcore_axis_name="s")
    sc_edge = functools.partial(
        pl.kernel,
        mesh=mesh,
        compiler_params=pltpu.CompilerParams(use_tc_tiling_on_sc=False),
        out_type=jax.ShapeDtypeStruct((2, N, EXT), f32),
        scratch_types=[
            pltpu.VMEM((CH,), jnp.int32),
            pltpu.VMEM((CH,), jnp.int32),
            pltpu.VMEM((CH, EXT), f32),
            pltpu.VMEM((CH, 16), f32),
            pltpu.VMEM((CH,), jnp.int32),
            pltpu.VMEM((CH,), jnp.int32),
            pltpu.VMEM((CH, EXT), f32),
            pltpu.VMEM((CH, 16), f32),
            pltpu.VMEM((CH,), jnp.int32),
            pltpu.VMEM((CH,), jnp.int32),
            pltpu.VMEM((ZROWS, EXT), f32),
            pltpu.VMEM_SHARED((N, EXT), f32),
            pltpu.SemaphoreType.DMA,
            pltpu.SemaphoreType.DMA,
            pltpu.SemaphoreType.DMA,
            pltpu.SemaphoreType.DMA,
            pltpu.SemaphoreType.DMA,
            pltpu.SemaphoreType.DMA,
        ],
    )(_sc_edge_body)
    acc = sc_edge(hs_ext, er_pad, edge_index)

    out = pl.pallas_call(
        _tc_post_body,
        grid=grid,
        in_specs=[
            pl.BlockSpec((2, blk, EXT), lambda i: (0, i, 0)),
            pl.BlockSpec((H, D), lambda i: (0, 0)),
        ],
        out_specs=pl.BlockSpec((blk, D), lambda i: (i, 0)),
        out_shape=jax.ShapeDtypeStruct((N, D), f32),
    )(acc, R)
    return out
